# baseline (speedup 1.0000x reference)
import jax
import jax.numpy as jnp
from jax import lax
from jax.experimental import pallas as pl
from jax.experimental.pallas import tpu as pltpu
from jax.experimental.pallas import tpu_sc as plsc

_N, _J, _K = 16384, 200, 5
_BLOCK_I = 4096


def _body(t_ref, x_ref, o_ref):
    xb = x_ref[...] != 0
    for k in range(_K):
        o_ref[k * _J:(k + 1) * _J, :] = jnp.where(xb, t_ref[1, k], t_ref[0, k])


def _kernel_tc(x, table):
    xt = x.T
    o2 = pl.pallas_call(
        _body,
        grid=(_N // _BLOCK_I,),
        in_specs=[
            pl.BlockSpec(memory_space=pltpu.SMEM),
            pl.BlockSpec((_J, _BLOCK_I), lambda i: (0, i)),
        ],
        out_specs=pl.BlockSpec((_J * _K, _BLOCK_I), lambda i: (0, i)),
        out_shape=jax.ShapeDtypeStruct((_J * _K, _N), jnp.float32),
    )(table, xt)
    return o2.reshape(_K, _J, _N).transpose(2, 1, 0)


_NW = 32
_CW = 256
_GDN = lax.GatherDimensionNumbers(
    offset_dims=(), collapsed_slice_dims=(0,), start_index_map=(0,))


def _sc_body(tflat_hbm, xt_hbm, o_hbm, tabs_v, xbuf_v, obuf_v):
    wid = lax.axis_index("s") * 2 + lax.axis_index("c")
    pltpu.sync_copy(tflat_hbm, tabs_v)
    tabs = tabs_v[...]
    for p in range(2):
        base = wid * 512 + p * _CW
        pltpu.sync_copy(xt_hbm.at[:, pl.ds(base, _CW)], xbuf_v)
        for k in range(_K):

            def jbody(j, _, k=k):
                for v in range(_CW // 16):
                    xv = xbuf_v[j, pl.ds(v * 16, 16)]
                    idx = (xv * _K + k).reshape(16, 1)
                    obuf_v[j, pl.ds(v * 16, 16)] = lax.gather(
                        tabs, idx, _GDN, (1,),
                        mode=lax.GatherScatterMode.PROMISE_IN_BOUNDS)
                return 0

            lax.fori_loop(0, _J, jbody, 0)
            pltpu.sync_copy(
                obuf_v, o_hbm.at[pl.ds(k * _J, _J), pl.ds(base, _CW)])


def _kernel_sc(x, table):
    xt = x.T
    mesh = plsc.VectorSubcoreMesh(core_axis_name="c", subcore_axis_name="s")
    tflat = jnp.pad(table.reshape(2 * _K), (0, 16 - 2 * _K))
    o2 = pl.kernel(
        _sc_body,
        out_type=jax.ShapeDtypeStruct((_J * _K, _N), jnp.float32),
        mesh=mesh,
        scratch_types=[
            pltpu.VMEM((16,), jnp.float32),
            pltpu.VMEM((_J, _CW), jnp.int32),
            pltpu.VMEM((_J, _CW), jnp.float32),
        ],
    )(tflat, xt)
    return o2.reshape(_K, _J, _N).transpose(2, 1, 0)


def kernel(x, table):
    return _kernel_sc(x, table)


# speedup vs baseline: 2.6474x; 2.6474x over previous
import jax
import jax.numpy as jnp
from jax.experimental import pallas as pl
from jax.experimental.pallas import tpu as pltpu

_N, _J, _K = 16384, 200, 5
_BLOCK_I = 4096


def _body(t_ref, x_ref, o_ref):
    xb = x_ref[...] != 0
    for k in range(_K):
        o_ref[k * _J:(k + 1) * _J, :] = jnp.where(xb, t_ref[1, k], t_ref[0, k])


def kernel(x, table):
    xt = x.T
    o2 = pl.pallas_call(
        _body,
        grid=(_N // _BLOCK_I,),
        in_specs=[
            pl.BlockSpec(memory_space=pltpu.SMEM),
            pl.BlockSpec((_J, _BLOCK_I), lambda i: (0, i)),
        ],
        out_specs=pl.BlockSpec((_J * _K, _BLOCK_I), lambda i: (0, i)),
        out_shape=jax.ShapeDtypeStruct((_J * _K, _N), jnp.float32),
    )(table, xt)
    return o2.reshape(_K, _J, _N).transpose(2, 1, 0)
